# baseline (speedup 1.0000x reference)
import jax
import jax.numpy as jnp
from jax import lax
from jax.experimental import pallas as pl
from jax.experimental.pallas import tpu as pltpu
from jax.experimental.pallas import tpu_sc as plsc

BATCH = 16384
EMB = 64
CDIM = 128
NC = 2
NS = 16
NW = NC * NS
BPW = BATCH // NW
CHUNK = 128
NCHUNK = BPW // CHUNK
NSLOT = 4


def _sc_gather_body(u_tab, c_tab, iu_hbm, ii_hbm, out_u, out_c,
                    idx_u, idx_c, ubuf, cbuf,
                    sg0, sg1, sg2, sg3, sw0, sw1, sw2, sw3):
    sem_g = [sg0, sg1, sg2, sg3]
    sem_w = [sw0, sw1, sw2, sw3]
    wid = lax.axis_index("s") * NC + lax.axis_index("c")
    base = wid * BPW
    rbase = wid * NCHUNK
    pltpu.sync_copy(iu_hbm.at[pl.ds(rbase, NCHUNK)], idx_u)
    pltpu.sync_copy(ii_hbm.at[pl.ds(rbase, NCHUNK)], idx_c)
    tasks = []
    for j in range(NCHUNK):
        tasks.append((u_tab, idx_u, out_u, ubuf, j))
        tasks.append((c_tab, idx_c, out_c, cbuf, j))
    nt = len(tasks)
    gh = [None] * nt
    wh = [None] * nt

    def start_wb(k):
        tab, idx, out, buf, j = tasks[k]
        s = (k // 2) % (NSLOT // 2)
        return pltpu.async_copy(
            buf.at[s], out.at[pl.ds(base + j * CHUNK, CHUNK)],
            sem_w[k % NSLOT])

    for k, (tab, idx, out, buf, j) in enumerate(tasks):
        s = (k // 2) % (NSLOT // 2)
        if k >= NSLOT:
            wh[k - NSLOT].wait()
        gh[k] = pltpu.async_copy(tab.at[idx.at[j]], buf.at[s], sem_g[k % NSLOT])
        if k >= 1:
            gh[k - 1].wait()
            wh[k - 1] = start_wb(k - 1)
    gh[nt - 1].wait()
    wh[nt - 1] = start_wb(nt - 1)
    for k in range(nt - NSLOT, nt):
        wh[k].wait()


@jax.jit
def _sc_gather(user_emb, item_content, iu, ii):
    mesh = plsc.VectorSubcoreMesh(core_axis_name="c", subcore_axis_name="s")
    return pl.kernel(
        _sc_gather_body,
        out_type=(
            jax.ShapeDtypeStruct((BATCH, EMB), jnp.float32),
            jax.ShapeDtypeStruct((BATCH, CDIM), jnp.float32),
        ),
        mesh=mesh,
        scratch_types=[
            pltpu.VMEM((NCHUNK, CHUNK), jnp.int32),
            pltpu.VMEM((NCHUNK, CHUNK), jnp.int32),
            pltpu.VMEM((NSLOT // 2, CHUNK, EMB), jnp.float32),
            pltpu.VMEM((NSLOT // 2, CHUNK, CDIM), jnp.float32),
        ] + [pltpu.SemaphoreType.DMA] * (2 * NSLOT),
        compiler_params=pltpu.CompilerParams(use_tc_tiling_on_sc=False),
    )(user_emb, item_content, iu, ii)


def _tc_body(u_ref, c_ref, w_ref, b_ref, o_ref):
    meta = lax.dot_general(c_ref[...], w_ref[...],
                           (((1,), (0,)), ((), ())),
                           preferred_element_type=jnp.float32)
    meta = meta + b_ref[...]
    o_ref[...] = jnp.sum(u_ref[...] * meta, axis=1)[None, None, :]


@jax.jit
def _tc_compute(u_g, c_g, Wt5, b5):
    blk = 2048
    grid = BATCH // blk
    out = pl.pallas_call(
        _tc_body,
        grid=(grid,),
        in_specs=[
            pl.BlockSpec((blk, EMB), lambda i: (i, 0)),
            pl.BlockSpec((blk, CDIM), lambda i: (i, 0)),
            pl.BlockSpec((CDIM, EMB), lambda i: (0, 0)),
            pl.BlockSpec((1, EMB), lambda i: (0, 0)),
        ],
        out_specs=pl.BlockSpec((1, 1, blk), lambda i: (i, 0, 0)),
        out_shape=jax.ShapeDtypeStruct((grid, 1, blk), jnp.float32),
    )(u_g, c_g, Wt5, b5)
    return out.reshape(BATCH)


def kernel(batch_u, batch_i, user_emb, item_emb, item_content, W, b):
    iu = batch_u.astype(jnp.int32).reshape(BATCH // CHUNK, CHUNK)
    ii = batch_i.astype(jnp.int32).reshape(BATCH // CHUNK, CHUNK)
    u_g, c_g = _sc_gather(user_emb, item_content, iu, ii)
    Wt5 = W.T / 5.0
    b5 = (b / 5.0).reshape(1, EMB)
    return _tc_compute(u_g, c_g, Wt5, b5)
